# Initial kernel scaffold; baseline (speedup 1.0000x reference)
import jax
import jax.numpy as jnp
from jax.experimental import pallas as pl


def kernel(inputs, gamma, beta, inner_bias, domain):
    raise NotImplementedError("write your pallas kernel here")


# speedup vs baseline: 1.4910x; 1.4910x over previous
import jax
import jax.numpy as jnp
from jax.experimental import pallas as pl
from jax.experimental.pallas import tpu as pltpu

_BATCH = 16384
_DIM = 128
_TABLE_ROWS = 6
_EPS = 1e-6
_BLOCK_ROWS = 1024


def _ln_bias_kernel(dom_ref, x_ref, gamma_ref, beta_ref, table_ref, o_ref):
    x = x_ref[...]
    mean = jnp.mean(x, axis=1, keepdims=True)
    xc = x - mean
    var = jnp.mean(xc * xc, axis=1, keepdims=True)
    inv = jax.lax.rsqrt(var + _EPS)
    d = dom_ref[0]
    bias = beta_ref[...] + table_ref[pl.ds(d, 1), :]
    o_ref[...] = xc * inv * gamma_ref[...] + bias


def kernel(inputs, gamma, beta, inner_bias, domain):
    dom = jnp.asarray(domain, dtype=jnp.int32).reshape((1,))
    gamma2 = gamma.reshape(1, _DIM)
    beta2 = beta.reshape(1, _DIM)
    grid = (_BATCH // _BLOCK_ROWS,)
    return pl.pallas_call(
        _ln_bias_kernel,
        grid=grid,
        in_specs=[
            pl.BlockSpec(memory_space=pltpu.SMEM),
            pl.BlockSpec((_BLOCK_ROWS, _DIM), lambda i: (i, 0)),
            pl.BlockSpec((1, _DIM), lambda i: (0, 0)),
            pl.BlockSpec((1, _DIM), lambda i: (0, 0)),
            pl.BlockSpec((_TABLE_ROWS, _DIM), lambda i: (0, 0)),
        ],
        out_specs=pl.BlockSpec((_BLOCK_ROWS, _DIM), lambda i: (i, 0)),
        out_shape=jax.ShapeDtypeStruct((_BATCH, _DIM), jnp.float32),
        compiler_params=pltpu.CompilerParams(
            dimension_semantics=("parallel",),
        ),
    )(dom, inputs, gamma2, beta2, inner_bias)


# speedup vs baseline: 1.9477x; 1.3063x over previous
import jax
import jax.numpy as jnp
from jax.experimental import pallas as pl
from jax.experimental.pallas import tpu as pltpu

_BATCH = 16384
_DIM = 128
_TABLE_ROWS = 6
_EPS = 1e-6
_BLOCK_ROWS = 2048


def _ln_bias_kernel(dom_ref, x_ref, gamma_ref, beta_ref, table_ref, o_ref):
    x = x_ref[...]
    mean = jnp.mean(x, axis=1, keepdims=True)
    xc = x - mean
    var = jnp.mean(xc * xc, axis=1, keepdims=True)
    inv = jax.lax.rsqrt(var + _EPS)
    d = dom_ref[0]
    bias = beta_ref[...] + table_ref[pl.ds(d, 1), :]
    o_ref[...] = xc * inv * gamma_ref[...] + bias


def kernel(inputs, gamma, beta, inner_bias, domain):
    dom = jnp.asarray(domain, dtype=jnp.int32).reshape((1,))
    gamma2 = gamma.reshape(1, _DIM)
    beta2 = beta.reshape(1, _DIM)
    grid = (_BATCH // _BLOCK_ROWS,)
    return pl.pallas_call(
        _ln_bias_kernel,
        grid=grid,
        in_specs=[
            pl.BlockSpec(memory_space=pltpu.SMEM),
            pl.BlockSpec((_BLOCK_ROWS, _DIM), lambda i: (i, 0)),
            pl.BlockSpec((1, _DIM), lambda i: (0, 0)),
            pl.BlockSpec((1, _DIM), lambda i: (0, 0)),
            pl.BlockSpec((_TABLE_ROWS, _DIM), lambda i: (0, 0)),
        ],
        out_specs=pl.BlockSpec((_BLOCK_ROWS, _DIM), lambda i: (i, 0)),
        out_shape=jax.ShapeDtypeStruct((_BATCH, _DIM), jnp.float32),
        compiler_params=pltpu.CompilerParams(
            dimension_semantics=("parallel",),
        ),
    )(dom, inputs, gamma2, beta2, inner_bias)
